# baseline (speedup 1.0000x reference)
import jax
import jax.numpy as jnp
from jax.experimental import pallas as pl

S_BLK = 1024


def _geno_block(xt_ref, a_ref, p_ref, o_ref):
    p = p_ref[...]
    a = a_ref[...]
    xt = xt_ref[...]
    for bi in range(xt.shape[0]):
        y = jax.lax.dot_general(
            xt[bi], a,
            dimension_numbers=(((0,), (0,)), ((), ())),
            preferred_element_type=jnp.float32,
        )
        o_ref[bi] = y + p


@jax.jit
def kernel(x, allele_embedding, position_embedding):
    B, S, N = x.shape
    D = allele_embedding.shape[1]
    xt = x.transpose(0, 2, 1)
    grid = ((S + S_BLK - 1) // S_BLK,)
    out = pl.pallas_call(
        _geno_block,
        grid=grid,
        in_specs=[
            pl.BlockSpec((B, N, S_BLK), lambda i: (0, 0, i)),
            pl.BlockSpec((N, D), lambda i: (0, 0)),
            pl.BlockSpec((S_BLK, D), lambda i: (i, 0)),
        ],
        out_specs=pl.BlockSpec((B, S_BLK, D), lambda i: (0, i, 0)),
        out_shape=jax.ShapeDtypeStruct((B, S, D), jnp.float32),
    )(xt, allele_embedding, position_embedding)
    return out


# speedup vs baseline: 1.0161x; 1.0161x over previous
import jax
import jax.numpy as jnp
from jax.experimental import pallas as pl

S_BLK = 1280


def _geno_block(xt_ref, a_ref, p_ref, o_ref):
    p = p_ref[...]
    a = a_ref[...]
    xt = xt_ref[...]
    for bi in range(xt.shape[0]):
        y = jax.lax.dot_general(
            xt[bi], a,
            dimension_numbers=(((0,), (0,)), ((), ())),
            preferred_element_type=jnp.float32,
        )
        o_ref[bi] = y + p


@jax.jit
def kernel(x, allele_embedding, position_embedding):
    B, S, N = x.shape
    D = allele_embedding.shape[1]
    xt = x.transpose(0, 2, 1)
    grid = ((S + S_BLK - 1) // S_BLK,)
    out = pl.pallas_call(
        _geno_block,
        grid=grid,
        in_specs=[
            pl.BlockSpec((B, N, S_BLK), lambda i: (0, 0, i)),
            pl.BlockSpec((N, D), lambda i: (0, 0)),
            pl.BlockSpec((S_BLK, D), lambda i: (i, 0)),
        ],
        out_specs=pl.BlockSpec((B, S_BLK, D), lambda i: (0, i, 0)),
        out_shape=jax.ShapeDtypeStruct((B, S, D), jnp.float32),
    )(xt, allele_embedding, position_embedding)
    return out


# speedup vs baseline: 1.0196x; 1.0034x over previous
import jax
import jax.numpy as jnp
from jax.experimental import pallas as pl

S_BLK = 1408


def _geno_block(xt_ref, a_ref, p_ref, o_ref):
    p = p_ref[...]
    a = a_ref[...]
    xt = xt_ref[...]
    for bi in range(xt.shape[0]):
        y = jax.lax.dot_general(
            xt[bi], a,
            dimension_numbers=(((0,), (0,)), ((), ())),
            preferred_element_type=jnp.float32,
        )
        o_ref[bi] = y + p


@jax.jit
def kernel(x, allele_embedding, position_embedding):
    B, S, N = x.shape
    D = allele_embedding.shape[1]
    xt = x.transpose(0, 2, 1)
    grid = ((S + S_BLK - 1) // S_BLK,)
    out = pl.pallas_call(
        _geno_block,
        grid=grid,
        in_specs=[
            pl.BlockSpec((B, N, S_BLK), lambda i: (0, 0, i)),
            pl.BlockSpec((N, D), lambda i: (0, 0)),
            pl.BlockSpec((S_BLK, D), lambda i: (i, 0)),
        ],
        out_specs=pl.BlockSpec((B, S_BLK, D), lambda i: (0, i, 0)),
        out_shape=jax.ShapeDtypeStruct((B, S, D), jnp.float32),
    )(xt, allele_embedding, position_embedding)
    return out
